# Initial kernel scaffold; baseline (speedup 1.0000x reference)
import jax
import jax.numpy as jnp
from jax.experimental import pallas as pl


def kernel(x0, x1, indices):
    raise NotImplementedError("write your pallas kernel here")


# speedup vs baseline: 3.6745x; 3.6745x over previous
import functools

import jax
import jax.numpy as jnp
from jax import lax
from jax.experimental import pallas as pl
from jax.experimental.pallas import tpu as pltpu
from jax.experimental.pallas import tpu_sc as plsc


def kernel(x0, x1, indices):
    E, D = x1.shape
    info = plsc.get_sparse_core_info()
    NC, NS = info.num_cores, info.num_subcores
    NW = NC * NS
    assert E % NW == 0
    e_per_w = E // NW
    B = 40
    assert e_per_w % (2 * B) == 0
    niter = e_per_w // B

    mesh = plsc.VectorSubcoreMesh(core_axis_name="c", subcore_axis_name="s")

    @functools.partial(
        pl.kernel,
        mesh=mesh,
        out_type=jax.ShapeDtypeStruct((E, D), jnp.float32),
        scratch_types=[
            pltpu.VMEM((e_per_w,), jnp.int32),
            pltpu.VMEM((B, D), jnp.float32),
            pltpu.VMEM((B, D), jnp.float32),
            pltpu.VMEM((B, D), jnp.float32),
            pltpu.VMEM((B, D), jnp.float32),
            pltpu.VMEM((B, D), jnp.float32),
            pltpu.VMEM((B, D), jnp.float32),
            pltpu.SemaphoreType.DMA,
            pltpu.SemaphoreType.DMA,
            pltpu.SemaphoreType.DMA,
            pltpu.SemaphoreType.DMA,
            pltpu.SemaphoreType.DMA,
            pltpu.SemaphoreType.DMA,
        ],
    )
    def run(x0_hbm, x1_hbm, idx_hbm, out_hbm,
            idx_v, w0, w1, y0, y1, o0, o1,
            g0, g1, p0, p1, q0, q1):
        wid = lax.axis_index("s") * NC + lax.axis_index("c")
        base = wid * e_per_w
        pltpu.sync_copy(idx_hbm.at[pl.ds(base, e_per_w)], idx_v)

        wbufs = (w0, w1)
        ybufs = (y0, y1)
        obufs = (o0, o1)
        gsems = (g0, g1)
        xsems = (p0, p1)
        osems = (q0, q1)

        def issue_inputs(i, s):
            pltpu.async_copy(
                x0_hbm.at[idx_v.at[pl.ds(i * B, B)]], wbufs[s], gsems[s])
            pltpu.async_copy(
                x1_hbm.at[pl.ds(base + i * B, B)], ybufs[s], xsems[s])

        issue_inputs(0, 0)
        issue_inputs(1, 1)

        def step(i, s):
            w, y, o = wbufs[s], ybufs[s], obufs[s]
            pltpu.make_async_copy(x1_hbm.at[pl.ds(0, B)], w, gsems[s]).wait()
            pltpu.make_async_copy(x1_hbm.at[pl.ds(0, B)], y, xsems[s]).wait()

            @pl.when(i >= 2)
            def _():
                pltpu.make_async_copy(o, out_hbm.at[pl.ds(0, B)], osems[s]).wait()

            def row(r, carry):
                for c in range(D // 16):
                    sl = pl.ds(c * 16, 16)
                    o[r, sl] = w[r, sl] * y[r, sl]
                return carry

            lax.fori_loop(0, B, row, 0)

            pltpu.async_copy(o, out_hbm.at[pl.ds(base + i * B, B)], osems[s])

            @pl.when(i + 2 < niter)
            def _():
                issue_inputs(i + 2, s)

        def outer(g, carry):
            step(2 * g, 0)
            step(2 * g + 1, 1)
            return carry

        lax.fori_loop(0, niter // 2, outer, 0)

        pltpu.make_async_copy(o0, out_hbm.at[pl.ds(0, B)], osems[0]).wait()
        pltpu.make_async_copy(o1, out_hbm.at[pl.ds(0, B)], osems[1]).wait()

    return run(x0, x1, indices)


# speedup vs baseline: 4.2647x; 1.1606x over previous
import functools

import jax
import jax.numpy as jnp
from jax import lax
from jax.experimental import pallas as pl
from jax.experimental.pallas import tpu as pltpu
from jax.experimental.pallas import tpu_sc as plsc


def kernel(x0, x1, indices):
    E, D = x1.shape
    info = plsc.get_sparse_core_info()
    NC, NS = info.num_cores, info.num_subcores
    NW = NC * NS
    assert E % NW == 0
    e_per_w = E // NW
    B = 80
    assert e_per_w % B == 0
    niter = e_per_w // B

    mesh = plsc.VectorSubcoreMesh(core_axis_name="c", subcore_axis_name="s")

    @functools.partial(
        pl.kernel,
        mesh=mesh,
        out_type=jax.ShapeDtypeStruct((E, D), jnp.float32),
        scratch_types=[
            pltpu.VMEM((e_per_w,), jnp.int32),
            pltpu.VMEM((B, D), jnp.float32),
            pltpu.VMEM((B, D), jnp.float32),
            pltpu.VMEM((B, D), jnp.float32),
            pltpu.VMEM((B, D), jnp.float32),
            pltpu.VMEM((B, D), jnp.float32),
            pltpu.VMEM((B, D), jnp.float32),
            pltpu.SemaphoreType.DMA,
            pltpu.SemaphoreType.DMA,
            pltpu.SemaphoreType.DMA,
            pltpu.SemaphoreType.DMA,
            pltpu.SemaphoreType.DMA,
            pltpu.SemaphoreType.DMA,
        ],
    )
    def run(x0_hbm, x1_hbm, idx_hbm, out_hbm,
            idx_v, w0, w1, y0, y1, o0, o1,
            g0, g1, p0, p1, q0, q1):
        wid = lax.axis_index("s") * NC + lax.axis_index("c")
        base = wid * e_per_w
        pltpu.sync_copy(idx_hbm.at[pl.ds(base, e_per_w)], idx_v)

        wbufs = (w0, w1)
        ybufs = (y0, y1)
        obufs = (o0, o1)
        gsems = (g0, g1)
        xsems = (p0, p1)
        osems = (q0, q1)

        def issue_inputs(i, s):
            pltpu.async_copy(
                x0_hbm.at[idx_v.at[pl.ds(i * B, B)]], wbufs[s], gsems[s])
            pltpu.async_copy(
                x1_hbm.at[pl.ds(base + i * B, B)], ybufs[s], xsems[s])

        issue_inputs(0, 0)
        issue_inputs(1, 1)

        def step(i, s):
            w, y, o = wbufs[s], ybufs[s], obufs[s]
            pltpu.make_async_copy(x1_hbm.at[pl.ds(0, B)], w, gsems[s]).wait()
            pltpu.make_async_copy(x1_hbm.at[pl.ds(0, B)], y, xsems[s]).wait()

            @pl.when(i >= 2)
            def _():
                pltpu.make_async_copy(o, out_hbm.at[pl.ds(0, B)], osems[s]).wait()

            @plsc.parallel_loop(0, B, unroll=4)
            def row(r):
                for c in range(D // 16):
                    sl = pl.ds(c * 16, 16)
                    o[r, sl] = w[r, sl] * y[r, sl]

            pltpu.async_copy(o, out_hbm.at[pl.ds(base + i * B, B)], osems[s])

            @pl.when(i + 2 < niter)
            def _():
                issue_inputs(i + 2, s)

        def outer(g, carry):
            step(2 * g, 0)
            step(2 * g + 1, 1)
            return carry

        lax.fori_loop(0, niter // 2, outer, 0)
        if niter % 2:
            step(niter - 1, 0)

        pltpu.make_async_copy(o0, out_hbm.at[pl.ds(0, B)], osems[0]).wait()
        pltpu.make_async_copy(o1, out_hbm.at[pl.ds(0, B)], osems[1]).wait()

    return run(x0, x1, indices)
